# baseline (speedup 1.0000x reference)
import functools

import jax
import jax.numpy as jnp
from jax import lax
from jax.experimental import pallas as pl
from jax.experimental.pallas import tpu as pltpu
from jax.experimental.pallas import tpu_sc as plsc

_EPS = 1e-8
_TOPK = 5
_NEG_INF = float("-inf")
_IMAX = 2147483647


def _topk_body(k_total, tk, x_ref, fb_ref, vals_ref, idx_ref):
    i = pl.program_id(0)
    q = x_ref.shape[0]
    d = x_ref.shape[1]

    x = x_ref[...]
    fb = fb_ref[...]
    ones = jnp.ones((1, d), dtype=jnp.float32)
    bsq = lax.dot_general(ones, fb * fb, (((1,), (1,)), ((), ())),
                          preferred_element_type=jnp.float32,
                          precision=lax.Precision.HIGHEST)
    inv_bn = 1.0 / jnp.maximum(jnp.sqrt(bsq), _EPS)
    dots = lax.dot_general(x, fb, (((1,), (1,)), ((), ())),
                           preferred_element_type=jnp.float32,
                           precision=lax.Precision.DEFAULT)
    cos = dots * inv_bn
    colid = lax.broadcasted_iota(jnp.int32, (q, tk), 1) + i * tk
    cos = jnp.where(colid < k_total, cos, _NEG_INF)

    @pl.when(i == 0)
    def _():
        vals_ref[...] = jnp.full((q, 128), _NEG_INF, jnp.float32)
        idx_ref[...] = jnp.full((q, 128), _IMAX, jnp.int32)

    half = tk // 2
    a, b = cos[:, :half], cos[:, half:]
    ia, ib = colid[:, :half], colid[:, half:]
    c = a >= b
    hi = jnp.where(c, a, b)
    hii = jnp.where(c, ia, ib)
    lov = jnp.where(c, b, a)
    lovi = jnp.where(c, ib, ia)
    rv = vals_ref[...]
    ri = idx_ref[...]
    lane = lax.broadcasted_iota(jnp.int32, (q, 128), 1)
    newv = jnp.full((q, 128), _NEG_INF, jnp.float32)
    newi = jnp.full((q, 128), _IMAX, jnp.int32)
    for j in range(_TOPK):
        m = jnp.maximum(jnp.max(hi, axis=1, keepdims=True),
                        jnp.max(rv, axis=1, keepdims=True))
        sel = jnp.minimum(
            jnp.min(jnp.where(hi == m, hii, _IMAX), axis=1, keepdims=True),
            jnp.min(jnp.where(rv == m, ri, _IMAX), axis=1, keepdims=True))
        mask_h = hii == sel
        mask_r = ri == sel
        hi = jnp.where(mask_h, lov, hi)
        hii = jnp.where(mask_h, lovi, hii)
        lov = jnp.where(mask_h, _NEG_INF, lov)
        rv = jnp.where(mask_r, _NEG_INF, rv)
        newv = jnp.where(lane == j, m, newv)
        newi = jnp.where(lane == j, sel, newi)
    vals_ref[...] = newv
    idx_ref[...] = newi


def _run_topk(x, feature_bank, tk=2048):
    q, d = x.shape
    k_total = feature_bank.shape[0]
    nt = -(-k_total // tk)
    return pl.pallas_call(
        functools.partial(_topk_body, k_total, tk),
        grid=(nt,),
        in_specs=[
            pl.BlockSpec((q, d), lambda i: (0, 0)),
            pl.BlockSpec((tk, d), lambda i: (i, 0)),
        ],
        out_specs=[
            pl.BlockSpec((q, 128), lambda i: (0, 0)),
            pl.BlockSpec((q, 128), lambda i: (0, 0)),
        ],
        out_shape=[
            jax.ShapeDtypeStruct((q, 128), jnp.float32),
            jax.ShapeDtypeStruct((q, 128), jnp.int32),
        ],
        compiler_params=pltpu.CompilerParams(
            dimension_semantics=("arbitrary",)),
    )(x, feature_bank)


def _make_sc_gather(v, d, b):
    info = plsc.get_sparse_core_info()
    nc, ns = info.num_cores, info.num_subcores
    nw = nc * ns
    b_per_w = b // nw
    mesh = plsc.VectorSubcoreMesh(core_axis_name="c", subcore_axis_name="s")

    @functools.partial(
        pl.kernel, mesh=mesh,
        out_type=jax.ShapeDtypeStruct((b, d), jnp.float32),
        scratch_types=[
            pltpu.VMEM((b_per_w,), jnp.int32),
            pltpu.VMEM((b_per_w, d), jnp.float32),
            pltpu.SemaphoreType.DMA,
        ],
    )
    def gather_kernel(table_hbm, idx_hbm, out_hbm, idx_v, rows_v, sem):
        wid = lax.axis_index("s") * nc + lax.axis_index("c")
        base = wid * b_per_w
        pltpu.sync_copy(idx_hbm.at[pl.ds(base, b_per_w)], idx_v)
        pltpu.async_copy(table_hbm.at[idx_v], rows_v, sem).wait()
        pltpu.sync_copy(rows_v, out_hbm.at[pl.ds(base, b_per_w)])

    return gather_kernel


def _combine_body(x_ref, g_ref, vals_ref, out_ref):
    d = x_ref.shape[1]
    x = x_ref[...]
    x0 = x[0:1, :]
    xn0 = jnp.maximum(jnp.sqrt(jnp.sum(x0 * x0)), _EPS)
    v0 = vals_ref[0:1, :] / xn0
    lane = lax.broadcasted_iota(jnp.int32, (1, 128), 1)
    sims = jnp.where(lane < _TOPK, v0, 0.0)
    rates = jnp.sum(sims) / float(_TOPK)
    e = jnp.exp(v0)
    sexp = jnp.sum(e)
    w = (rates / sexp) * e
    acc = x * (1.0 - rates)
    for j in range(_TOPK):
        wj = jnp.sum(jnp.where(lane == j, w, 0.0))
        acc = acc + wj * g_ref[:, j * d:(j + 1) * d]
    out_ref[...] = acc


def _run_combine(x, gathered_flat, vals):
    q, d = x.shape
    return pl.pallas_call(
        _combine_body,
        in_specs=[
            pl.BlockSpec((q, d), lambda: (0, 0)),
            pl.BlockSpec((q, _TOPK * d), lambda: (0, 0)),
            pl.BlockSpec((q, 128), lambda: (0, 0)),
        ],
        out_specs=pl.BlockSpec((q, d), lambda: (0, 0)),
        out_shape=jax.ShapeDtypeStruct((q, d), jnp.float32),
    )(x, gathered_flat, vals)


def kernel(x, mask, feature_bank, top_k):
    q, d = x.shape
    k_total = feature_bank.shape[0]
    vals, idx = _run_topk(x, feature_bank)
    idx5 = idx[:, :_TOPK].reshape(-1)
    gathered = _make_sc_gather(k_total, d, q * _TOPK)(feature_bank, idx5)
    out = _run_combine(x, gathered.reshape(q, _TOPK * d), vals)
    return (out, jnp.asarray(k_total))


# speedup vs baseline: 1.1098x; 1.1098x over previous
import functools

import jax
import jax.numpy as jnp
from jax import lax
from jax.experimental import pallas as pl
from jax.experimental.pallas import tpu as pltpu
from jax.experimental.pallas import tpu_sc as plsc

_EPS = 1e-8
_TOPK = 5
_NEG_INF = float("-inf")
_IMAX = 2147483647


def _topk_body(k_total, tk, x_ref, fb_ref, vals_ref, idx_ref):
    i = pl.program_id(0)
    q = x_ref.shape[0]
    d = x_ref.shape[1]

    x = x_ref[...]
    fb = fb_ref[...]
    ones = jnp.ones((1, d), dtype=jnp.float32)
    bsq = lax.dot_general(ones, fb * fb, (((1,), (1,)), ((), ())),
                          preferred_element_type=jnp.float32,
                          precision=lax.Precision.HIGHEST)
    inv_bn = 1.0 / jnp.maximum(jnp.sqrt(bsq), _EPS)
    dots = lax.dot_general(x, fb, (((1,), (1,)), ((), ())),
                           preferred_element_type=jnp.float32,
                           precision=lax.Precision.DEFAULT)
    cos = dots * inv_bn
    colid = lax.broadcasted_iota(jnp.int32, (q, tk), 1) + i * tk
    cos = jnp.where(colid < k_total, cos, _NEG_INF)

    @pl.when(i == 0)
    def _():
        vals_ref[...] = jnp.full((q, 128), _NEG_INF, jnp.float32)
        idx_ref[...] = jnp.full((q, 128), _IMAX, jnp.int32)

    half = tk // 2
    a, b = cos[:, :half], cos[:, half:]
    ia, ib = colid[:, :half], colid[:, half:]
    c = a >= b
    ws = jnp.concatenate([jnp.where(c, a, b), vals_ref[...]], axis=1)
    wsi = jnp.concatenate([jnp.where(c, ia, ib), idx_ref[...]], axis=1)
    lov = jnp.concatenate(
        [jnp.where(c, b, a), jnp.full((q, 128), _NEG_INF, jnp.float32)],
        axis=1)
    lovi = jnp.concatenate(
        [jnp.where(c, ib, ia), jnp.full((q, 128), _IMAX, jnp.int32)],
        axis=1)
    lane = lax.broadcasted_iota(jnp.int32, (q, 128), 1)
    newv = jnp.full((q, 128), _NEG_INF, jnp.float32)
    newi = jnp.full((q, 128), _IMAX, jnp.int32)
    for j in range(_TOPK):
        m = jnp.max(ws, axis=1, keepdims=True)
        sel = jnp.min(jnp.where(ws == m, wsi, _IMAX), axis=1,
                      keepdims=True)
        maski = wsi == sel
        ws = jnp.where(maski, lov, ws)
        wsi = jnp.where(maski, lovi, wsi)
        lov = jnp.where(maski, _NEG_INF, lov)
        newv = jnp.where(lane == j, m, newv)
        newi = jnp.where(lane == j, sel, newi)
    vals_ref[...] = newv
    idx_ref[...] = newi


def _run_topk(x, feature_bank, tk=2048):
    q, d = x.shape
    k_total = feature_bank.shape[0]
    nt = -(-k_total // tk)
    return pl.pallas_call(
        functools.partial(_topk_body, k_total, tk),
        grid=(nt,),
        in_specs=[
            pl.BlockSpec((q, d), lambda i: (0, 0)),
            pl.BlockSpec((tk, d), lambda i: (i, 0)),
        ],
        out_specs=[
            pl.BlockSpec((q, 128), lambda i: (0, 0)),
            pl.BlockSpec((q, 128), lambda i: (0, 0)),
        ],
        out_shape=[
            jax.ShapeDtypeStruct((q, 128), jnp.float32),
            jax.ShapeDtypeStruct((q, 128), jnp.int32),
        ],
        compiler_params=pltpu.CompilerParams(
            dimension_semantics=("arbitrary",)),
    )(x, feature_bank)


def _make_sc_gather(v, d, b):
    info = plsc.get_sparse_core_info()
    nc, ns = info.num_cores, info.num_subcores
    nw = nc * ns
    b_per_w = b // nw
    mesh = plsc.VectorSubcoreMesh(core_axis_name="c", subcore_axis_name="s")

    @functools.partial(
        pl.kernel, mesh=mesh,
        out_type=jax.ShapeDtypeStruct((b, d), jnp.float32),
        scratch_types=[
            pltpu.VMEM((b_per_w,), jnp.int32),
            pltpu.VMEM((b_per_w, d), jnp.float32),
            pltpu.SemaphoreType.DMA,
        ],
    )
    def gather_kernel(table_hbm, idx_hbm, out_hbm, idx_v, rows_v, sem):
        wid = lax.axis_index("s") * nc + lax.axis_index("c")
        base = wid * b_per_w
        pltpu.sync_copy(idx_hbm.at[pl.ds(base, b_per_w)], idx_v)
        pltpu.async_copy(table_hbm.at[idx_v], rows_v, sem).wait()
        pltpu.sync_copy(rows_v, out_hbm.at[pl.ds(base, b_per_w)])

    return gather_kernel


def _combine_body(x_ref, g_ref, vals_ref, out_ref):
    d = x_ref.shape[1]
    x = x_ref[...]
    x0 = x[0:1, :]
    xn0 = jnp.maximum(jnp.sqrt(jnp.sum(x0 * x0)), _EPS)
    v0 = vals_ref[0:1, :] / xn0
    lane = lax.broadcasted_iota(jnp.int32, (1, 128), 1)
    sims = jnp.where(lane < _TOPK, v0, 0.0)
    rates = jnp.sum(sims) / float(_TOPK)
    e = jnp.exp(v0)
    sexp = jnp.sum(e)
    w = (rates / sexp) * e
    acc = x * (1.0 - rates)
    for j in range(_TOPK):
        wj = jnp.sum(jnp.where(lane == j, w, 0.0))
        acc = acc + wj * g_ref[:, j * d:(j + 1) * d]
    out_ref[...] = acc


def _run_combine(x, gathered_flat, vals):
    q, d = x.shape
    return pl.pallas_call(
        _combine_body,
        in_specs=[
            pl.BlockSpec((q, d), lambda: (0, 0)),
            pl.BlockSpec((q, _TOPK * d), lambda: (0, 0)),
            pl.BlockSpec((q, 128), lambda: (0, 0)),
        ],
        out_specs=pl.BlockSpec((q, d), lambda: (0, 0)),
        out_shape=jax.ShapeDtypeStruct((q, d), jnp.float32),
    )(x, gathered_flat, vals)


def kernel(x, mask, feature_bank, top_k):
    q, d = x.shape
    k_total = feature_bank.shape[0]
    vals, idx = _run_topk(x, feature_bank)
    idx5 = idx[:, :_TOPK].reshape(-1)
    gathered = _make_sc_gather(k_total, d, q * _TOPK)(feature_bank, idx5)
    out = _run_combine(x, gathered.reshape(q, _TOPK * d), vals)
    return (out, jnp.asarray(k_total))


# speedup vs baseline: 1.1880x; 1.0705x over previous
import functools

import jax
import jax.numpy as jnp
from jax import lax
from jax.experimental import pallas as pl
from jax.experimental.pallas import tpu as pltpu
from jax.experimental.pallas import tpu_sc as plsc

_EPS = 1e-8
_TOPK = 5
_NEG_INF = float("-inf")
_IMAX = 2147483647


def _topk_body(k_total, tk, x_ref, fb_ref, vals_ref, idx_ref):
    i = pl.program_id(0)
    q = x_ref.shape[0]
    d = x_ref.shape[1]

    x = x_ref[...]
    fb = fb_ref[...]
    ones = jnp.ones((1, d), dtype=jnp.float32)
    bsq = lax.dot_general(ones, fb * fb, (((1,), (1,)), ((), ())),
                          preferred_element_type=jnp.float32,
                          precision=lax.Precision.HIGHEST)
    inv_bn = 1.0 / jnp.maximum(jnp.sqrt(bsq), _EPS)
    dots = lax.dot_general(x, fb, (((1,), (1,)), ((), ())),
                           preferred_element_type=jnp.float32,
                           precision=lax.Precision.DEFAULT)
    cos = dots * inv_bn
    colid = lax.broadcasted_iota(jnp.int32, (q, tk), 1) + i * tk
    cos = jnp.where(colid < k_total, cos, _NEG_INF)

    @pl.when(i == 0)
    def _():
        vals_ref[...] = jnp.full((q, 128), _NEG_INF, jnp.float32)
        idx_ref[...] = jnp.full((q, 128), _IMAX, jnp.int32)

    half = tk // 2
    a, b = cos[:, :half], cos[:, half:]
    ia, ib = colid[:, :half], colid[:, half:]
    c = a >= b
    ws = jnp.concatenate([jnp.where(c, a, b), vals_ref[...]], axis=1)
    wsi = jnp.concatenate([jnp.where(c, ia, ib), idx_ref[...]], axis=1)
    lov = jnp.concatenate(
        [jnp.where(c, b, a), jnp.full((q, 128), _NEG_INF, jnp.float32)],
        axis=1)
    lovi = jnp.concatenate(
        [jnp.where(c, ib, ia), jnp.full((q, 128), _IMAX, jnp.int32)],
        axis=1)
    lane = lax.broadcasted_iota(jnp.int32, (q, 128), 1)
    newv = jnp.full((q, 128), _NEG_INF, jnp.float32)
    newi = jnp.full((q, 128), _IMAX, jnp.int32)
    for j in range(_TOPK):
        m = jnp.max(ws, axis=1, keepdims=True)
        mask = ws == m
        sel = jnp.min(jnp.where(mask, wsi, _IMAX), axis=1,
                      keepdims=True)
        ws = jnp.where(mask, lov, ws)
        wsi = jnp.where(mask, lovi, wsi)
        lov = jnp.where(mask, _NEG_INF, lov)
        newv = jnp.where(lane == j, m, newv)
        newi = jnp.where(lane == j, sel, newi)
    vals_ref[...] = newv
    idx_ref[...] = newi


def _run_topk(x, feature_bank, tk=4096):
    q, d = x.shape
    k_total = feature_bank.shape[0]
    nt = -(-k_total // tk)
    return pl.pallas_call(
        functools.partial(_topk_body, k_total, tk),
        grid=(nt,),
        in_specs=[
            pl.BlockSpec((q, d), lambda i: (0, 0)),
            pl.BlockSpec((tk, d), lambda i: (i, 0)),
        ],
        out_specs=[
            pl.BlockSpec((q, 128), lambda i: (0, 0)),
            pl.BlockSpec((q, 128), lambda i: (0, 0)),
        ],
        out_shape=[
            jax.ShapeDtypeStruct((q, 128), jnp.float32),
            jax.ShapeDtypeStruct((q, 128), jnp.int32),
        ],
        compiler_params=pltpu.CompilerParams(
            dimension_semantics=("arbitrary",)),
    )(x, feature_bank)


def _make_sc_gather(v, d, b):
    info = plsc.get_sparse_core_info()
    nc, ns = info.num_cores, info.num_subcores
    nw = nc * ns
    b_per_w = b // nw
    mesh = plsc.VectorSubcoreMesh(core_axis_name="c", subcore_axis_name="s")

    @functools.partial(
        pl.kernel, mesh=mesh,
        out_type=jax.ShapeDtypeStruct((b, d), jnp.float32),
        scratch_types=[
            pltpu.VMEM((b_per_w,), jnp.int32),
            pltpu.VMEM((b_per_w, d), jnp.float32),
            pltpu.SemaphoreType.DMA,
        ],
    )
    def gather_kernel(table_hbm, idx_hbm, out_hbm, idx_v, rows_v, sem):
        wid = lax.axis_index("s") * nc + lax.axis_index("c")
        base = wid * b_per_w
        pltpu.sync_copy(idx_hbm.at[pl.ds(base, b_per_w)], idx_v)
        pltpu.async_copy(table_hbm.at[idx_v], rows_v, sem).wait()
        pltpu.sync_copy(rows_v, out_hbm.at[pl.ds(base, b_per_w)])

    return gather_kernel


def _combine_body(x_ref, g_ref, vals_ref, out_ref):
    d = x_ref.shape[1]
    x = x_ref[...]
    x0 = x[0:1, :]
    xn0 = jnp.maximum(jnp.sqrt(jnp.sum(x0 * x0)), _EPS)
    v0 = vals_ref[0:1, :] / xn0
    lane = lax.broadcasted_iota(jnp.int32, (1, 128), 1)
    sims = jnp.where(lane < _TOPK, v0, 0.0)
    rates = jnp.sum(sims) / float(_TOPK)
    e = jnp.exp(v0)
    sexp = jnp.sum(e)
    w = (rates / sexp) * e
    acc = x * (1.0 - rates)
    for j in range(_TOPK):
        wj = jnp.sum(jnp.where(lane == j, w, 0.0))
        acc = acc + wj * g_ref[:, j * d:(j + 1) * d]
    out_ref[...] = acc


def _run_combine(x, gathered_flat, vals):
    q, d = x.shape
    return pl.pallas_call(
        _combine_body,
        in_specs=[
            pl.BlockSpec((q, d), lambda: (0, 0)),
            pl.BlockSpec((q, _TOPK * d), lambda: (0, 0)),
            pl.BlockSpec((q, 128), lambda: (0, 0)),
        ],
        out_specs=pl.BlockSpec((q, d), lambda: (0, 0)),
        out_shape=jax.ShapeDtypeStruct((q, d), jnp.float32),
    )(x, gathered_flat, vals)


def kernel(x, mask, feature_bank, top_k):
    q, d = x.shape
    k_total = feature_bank.shape[0]
    vals, idx = _run_topk(x, feature_bank)
    idx5 = idx[:, :_TOPK].reshape(-1)
    gathered = _make_sc_gather(k_total, d, q * _TOPK)(feature_bank, idx5)
    out = _run_combine(x, gathered.reshape(q, _TOPK * d), vals)
    return (out, jnp.asarray(k_total))
